# baseline (speedup 1.0000x reference)
import functools

import jax
import jax.numpy as jnp
from jax import lax
from jax.experimental import pallas as pl
from jax.experimental.pallas import tpu as pltpu
from jax.experimental.pallas import tpu_sc as plsc

N = 10000
D = 128
TEMP = 0.5
INV_T = 1.0 / TEMP
LAM = 0.001
LAMBDA_LOSS = 1.0
E = 320000

NPAD = 10240
NOUT = 10240
GARB = N
KS = 8

NC = 2
NSUB = 16
NW = NC * NSUB
CH = 79
EPW = CH * 128
ROWS_PT = NOUT // NSUB

BR = 1024
BC = 1024

_sc_mesh = functools.partial(
    plsc.VectorSubcoreMesh, core_axis_name="c", subcore_axis_name="s")


def _degree_pass(srcp, dstp):
    @functools.partial(
        pl.kernel,
        out_type=jax.ShapeDtypeStruct((NC, NOUT, 128), jnp.float32),
        mesh=_sc_mesh(),
        scratch_types=[
            pltpu.VMEM((CH, 128), jnp.int32),
            pltpu.VMEM((CH, 128), jnp.int32),
            pltpu.VMEM((128, 128), jnp.float32),
            pltpu.VMEM((16, 128), jnp.float32),
            pltpu.VMEM_SHARED((NOUT, 128), jnp.float32),
            pltpu.SemaphoreType.DMA,
        ],
    )
    def deg(srcp_hbm, dstp_hbm, out_hbm, idxs_v, idxd_v, e_v, zb_v, acc_sh,
            sd):
        c = lax.axis_index("c")
        s = lax.axis_index("s")
        wid = c * NSUB + s
        pltpu.sync_copy(srcp_hbm.at[wid], idxs_v)
        pltpu.sync_copy(dstp_hbm.at[wid], idxd_v)
        lane = lax.broadcasted_iota(jnp.int32, (16,), 0)
        e0 = jnp.where(lane == 0, 1.0, 0.0).astype(jnp.float32)
        e1 = jnp.where(lane == 1, 1.0, 0.0).astype(jnp.float32)
        z16 = jnp.zeros((16,), jnp.float32)

        def fill0(i, carry):
            e_v[i, pl.ds(0, 16)] = e0
            for j in range(1, 8):
                e_v[i, pl.ds(j * 16, 16)] = z16
            return carry

        def fill1(i, carry):
            e_v[i, pl.ds(0, 16)] = e1
            return carry

        lax.fori_loop(0, 128, fill0, 0)
        for i in range(16):
            for j in range(8):
                zb_v[i, pl.ds(j * 16, 16)] = z16
        tb = s * ROWS_PT

        def zloop(r, carry):
            pltpu.sync_copy(zb_v, acc_sh.at[pl.ds(tb + r * 16, 16)])
            return carry

        lax.fori_loop(0, ROWS_PT // 16, zloop, 0)
        plsc.subcore_barrier()

        def mk_body(idx_v):
            def body(j, carry):
                pltpu.async_copy(e_v, acc_sh.at[idx_v.at[j]], sd, add=True)

                @pl.when(j >= 8)
                def _():
                    pltpu.make_async_copy(
                        e_v, acc_sh.at[idx_v.at[j - 8]], sd).wait()

                return carry
            return body

        def drain(idx_v):
            def body(j, carry):
                pltpu.make_async_copy(
                    e_v, acc_sh.at[idx_v.at[j]], sd).wait()
                return carry
            lax.fori_loop(CH - 8, CH, body, 0)

        lax.fori_loop(0, CH, mk_body(idxs_v), 0)
        drain(idxs_v)
        lax.fori_loop(0, 128, fill1, 0)
        lax.fori_loop(0, CH, mk_body(idxd_v), 0)
        drain(idxd_v)
        plsc.subcore_barrier()
        pltpu.sync_copy(acc_sh.at[pl.ds(tb, ROWS_PT)],
                        out_hbm.at[c, pl.ds(tb, ROWS_PT)])

    return deg(srcp, dstp)


_AGG_SCRATCH = [
    pltpu.VMEM((CH, 128), jnp.int32),
    pltpu.VMEM((1, 128), jnp.int32),
    pltpu.VMEM((1, 128), jnp.int32),
    pltpu.VMEM((128, 128), jnp.float32),
    pltpu.VMEM((128, 128), jnp.float32),
    pltpu.VMEM((16, 128), jnp.float32),
    pltpu.VMEM_SHARED((NOUT, 128), jnp.float32),
    pltpu.SemaphoreType.DMA,
    pltpu.SemaphoreType.DMA,
    pltpu.SemaphoreType.DMA,
    pltpu.SemaphoreType.DMA,
    pltpu.SemaphoreType.DMA,
    pltpu.SemaphoreType.DMA,
]


def _zero_acc(zb_v, acc_sh, tb):
    z16 = jnp.zeros((16,), jnp.float32)
    for i in range(16):
        for j in range(8):
            zb_v[i, pl.ds(j * 16, 16)] = z16

    def zloop(r, carry):
        pltpu.sync_copy(zb_v, acc_sh.at[pl.ds(tb + r * 16, 16)])
        return carry

    lax.fori_loop(0, ROWS_PT // 16, zloop, 0)


def _agg_edges(table_hbm, srcp_hbm, wid, idxd_v, isl0_v, isl1_v, rows0_v,
               rows1_v, acc_sh, sg0, sg1, si0, si1, ss0, ss1):
    pltpu.sync_copy(srcp_hbm.at[wid, pl.ds(0, 1)], isl0_v)
    pltpu.async_copy(srcp_hbm.at[wid, pl.ds(1, 1)], isl1_v, si1)
    pltpu.async_copy(table_hbm.at[isl0_v.at[0]], rows0_v, sg0)

    def half(j, isl_a, isl_b, rows_a, rows_b, sg_a, sg_b, si_a, si_b,
             ss_a, ss_b):
        pltpu.make_async_copy(table_hbm.at[isl_a.at[0]], rows_a, sg_a).wait()

        @pl.when(j + 2 < CH)
        def _():
            pltpu.async_copy(srcp_hbm.at[wid, pl.ds(j + 2, 1)], isl_a, si_a)

        pltpu.async_copy(rows_a, acc_sh.at[idxd_v.at[j]], ss_a, add=True)

        @pl.when(j > 0)
        def _():
            pltpu.make_async_copy(
                rows_b, acc_sh.at[idxd_v.at[j - 1]], ss_b).wait()

        @pl.when(j + 1 < CH)
        def _():
            pltpu.make_async_copy(
                srcp_hbm.at[wid, pl.ds(j + 1, 1)], isl_b, si_b).wait()
            pltpu.async_copy(table_hbm.at[isl_b.at[0]], rows_b, sg_b)

    def body(j, carry):
        even = (j % 2) == 0

        @pl.when(even)
        def _():
            half(j, isl0_v, isl1_v, rows0_v, rows1_v, sg0, sg1, si0, si1,
                 ss0, ss1)

        @pl.when(jnp.logical_not(even))
        def _():
            half(j, isl1_v, isl0_v, rows1_v, rows0_v, sg1, sg0, si1, si0,
                 ss1, ss0)

        return carry

    lax.fori_loop(0, CH, body, 0)
    pltpu.make_async_copy(rows0_v, acc_sh.at[idxd_v.at[CH - 1]], ss0).wait()


def _rowagg_pass(table, srcp, dstp):
    @functools.partial(
        pl.kernel,
        out_type=jax.ShapeDtypeStruct((NC, NOUT, 128), jnp.float32),
        mesh=_sc_mesh(),
        scratch_types=list(_AGG_SCRATCH),
    )
    def agg(table_hbm, srcp_hbm, dstp_hbm, out_hbm, idxd_v, isl0_v, isl1_v,
            rows0_v, rows1_v, zb_v, acc_sh, sg0, sg1, si0, si1, ss0, ss1):
        c = lax.axis_index("c")
        s = lax.axis_index("s")
        wid = c * NSUB + s
        pltpu.sync_copy(dstp_hbm.at[wid], idxd_v)
        tb = s * ROWS_PT
        _zero_acc(zb_v, acc_sh, tb)
        plsc.subcore_barrier()
        _agg_edges(table_hbm, srcp_hbm, wid, idxd_v, isl0_v, isl1_v,
                   rows0_v, rows1_v, acc_sh, sg0, sg1, si0, si1, ss0, ss1)
        plsc.subcore_barrier()
        pltpu.sync_copy(acc_sh.at[pl.ds(tb, ROWS_PT)],
                        out_hbm.at[c, pl.ds(tb, ROWS_PT)])

    return agg(table, srcp, dstp)


def _dualagg_pass(table_a, table_b, srcp, dstp):
    @functools.partial(
        pl.kernel,
        out_type=jax.ShapeDtypeStruct((2, NC, NOUT, 128), jnp.float32),
        mesh=_sc_mesh(),
        scratch_types=list(_AGG_SCRATCH),
    )
    def agg(ta_hbm, tb_hbm, srcp_hbm, dstp_hbm, out_hbm, idxd_v, isl0_v,
            isl1_v, rows0_v, rows1_v, zb_v, acc_sh, sg0, sg1, si0, si1,
            ss0, ss1):
        c = lax.axis_index("c")
        s = lax.axis_index("s")
        wid = c * NSUB + s
        pltpu.sync_copy(dstp_hbm.at[wid], idxd_v)
        tb = s * ROWS_PT
        _zero_acc(zb_v, acc_sh, tb)
        plsc.subcore_barrier()
        _agg_edges(ta_hbm, srcp_hbm, wid, idxd_v, isl0_v, isl1_v,
                   rows0_v, rows1_v, acc_sh, sg0, sg1, si0, si1, ss0, ss1)
        plsc.subcore_barrier()
        pltpu.sync_copy(acc_sh.at[pl.ds(tb, ROWS_PT)],
                        out_hbm.at[0, c, pl.ds(tb, ROWS_PT)])
        plsc.subcore_barrier()
        _zero_acc(zb_v, acc_sh, tb)
        plsc.subcore_barrier()
        _agg_edges(tb_hbm, srcp_hbm, wid, idxd_v, isl0_v, isl1_v,
                   rows0_v, rows1_v, acc_sh, sg0, sg1, si0, si1, ss0, ss1)
        plsc.subcore_barrier()
        pltpu.sync_copy(acc_sh.at[pl.ds(tb, ROWS_PT)],
                        out_hbm.at[1, c, pl.ds(tb, ROWS_PT)])

    return agg(table_a, table_b, srcp, dstp)


def _tc_call(body, out_shape, *args):
    return pl.pallas_call(body, out_shape=out_shape)(*args)


def _gcn_in_body(x_ref, w_ref, degs_ref, out_ref):
    dego = degs_ref[0] + degs_ref[1]
    ns = lax.rsqrt(jnp.maximum(dego, 1.0))[:N]
    y = jnp.dot(x_ref[...], w_ref[...], preferred_element_type=jnp.float32)
    out_ref[0:N, :] = y * ns[:, None]
    out_ref[N:NPAD, :] = jnp.zeros((NPAD - N, D), jnp.float32)


def _target_body(x_ref, wt1_ref, bt1_ref, g_ref, be_ref, wt2_ref, bt2_ref,
                 wp_ref, bp_ref, out_ref):
    xv = x_ref[...]
    t = jnp.dot(xv, wt1_ref[...], preferred_element_type=jnp.float32)
    t = t + bt1_ref[...][None, :]
    mu = jnp.mean(t, axis=0)
    var = jnp.mean(t * t, axis=0) - mu * mu
    t = (t - mu[None, :]) * lax.rsqrt(var + 1e-5)[None, :]
    t = t * g_ref[...][None, :] + be_ref[...][None, :]
    t = jnp.maximum(t, 0.0)
    trans = jnp.dot(t, wt2_ref[...], preferred_element_type=jnp.float32)
    trans = trans + bt2_ref[...][None, :]
    p = jnp.dot(trans, wp_ref[...], preferred_element_type=jnp.float32)
    p = p + bp_ref[...][None, :]
    nrm = jnp.sqrt(jnp.sum(p * p, axis=1, keepdims=True))
    q = p / jnp.maximum(nrm, 1e-12)
    out_ref[0:N, :] = q
    out_ref[N:NPAD, :] = jnp.zeros((NPAD - N, D), jnp.float32)


def _gcn_mid_body(agg_ref, degs_ref, b0_ref, w1_ref, out_ref):
    dego = degs_ref[0] + degs_ref[1]
    degi = degs_ref[2] + degs_ref[3]
    ns = lax.rsqrt(jnp.maximum(dego, 1.0))[:N]
    nd = lax.rsqrt(jnp.maximum(degi, 1.0))[:N]
    a = (agg_ref[0] + agg_ref[1])[:N]
    h1 = jnp.maximum(a * nd[:, None] + b0_ref[...][None, :], 0.0)
    y = jnp.dot(h1, w1_ref[...], preferred_element_type=jnp.float32)
    out_ref[0:N, :] = y * ns[:, None]
    out_ref[N:NPAD, :] = jnp.zeros((NPAD - N, D), jnp.float32)


def _gcn_out_body(agg_ref, degs_ref, b1_ref, out_ref):
    degi = degs_ref[2] + degs_ref[3]
    nd = lax.rsqrt(jnp.maximum(degi, 1.0))[:N]
    a = (agg_ref[0] + agg_ref[1])[:N]
    h2 = a * nd[:, None] + b1_ref[...][None, :]
    nrm = jnp.sqrt(jnp.sum(h2 * h2, axis=1, keepdims=True))
    z = h2 / jnp.maximum(nrm, 1e-12)
    out_ref[0:N, :] = z
    out_ref[N:NPAD, :] = jnp.zeros((NPAD - N, D), jnp.float32)


def _nxn_body(zfull_ref, qfull_ref, zb_ref, v_ref):
    zb = zb_ref[...]

    def step(j, carry):
        a1, a2 = carry
        zc = zfull_ref[pl.ds(j * BC, BC), :]
        qc = qfull_ref[pl.ds(j * BC, BC), :]

        s1 = lax.dot_general(zb, zc, (((1,), (1,)), ((), ())),
                             preferred_element_type=jnp.float32)
        s2 = lax.dot_general(zb, qc, (((1,), (1,)), ((), ())),
                             preferred_element_type=jnp.float32)
        a1 = a1 + jnp.sum(jnp.exp(s1 * INV_T), axis=1)
        a2 = a2 + jnp.sum(jnp.exp(s2 * INV_T), axis=1)
        return a1, a2

    a1, a2 = lax.fori_loop(0, NPAD // BC, step,
                           (jnp.zeros((BR,), jnp.float32),
                            jnp.zeros((BR,), jnp.float32)))
    pad = float(NPAD - N)
    cc = a1 - pad
    aa = LAM * (a2 - pad)
    lane = lax.broadcasted_iota(jnp.int32, (BR, D), 1)
    kf = jnp.where(lane <= 7, lane + 1, 0).astype(jnp.float32)
    la = jnp.log(aa)[:, None]
    pw = jnp.exp(la * kf)
    v_ref[...] = jnp.where(
        lane <= 8, jnp.where(lane == 8, 1.0, pw),
        jnp.where(lane == 9, cc[:, None], 0.0))


def _nxn_powers(zpad, qpad):
    z16 = zpad.astype(jnp.bfloat16)
    q16 = qpad.astype(jnp.bfloat16)
    return pl.pallas_call(
        _nxn_body,
        grid=(NPAD // BR,),
        in_specs=[
            pl.BlockSpec((NPAD, D), lambda i: (0, 0)),
            pl.BlockSpec((NPAD, D), lambda i: (0, 0)),
            pl.BlockSpec((BR, D), lambda i: (i, 0)),
        ],
        out_specs=pl.BlockSpec((BR, D), lambda i: (i, 0)),
        out_shape=jax.ShapeDtypeStruct((NPAD, D), jnp.float32),
    )(z16, q16, z16)


def _loss_body(aggz_ref, aggp_ref, p_ref, z_ref, q_ref, out_ref):
    aggz = (aggz_ref[0] + aggz_ref[1])[:N]
    aggp = (aggp_ref[0] + aggp_ref[1])[:N, 0:16]
    pv = p_ref[...][:N]
    zv = z_ref[...][:N]
    qv = q_ref[...][:N]
    aggz = aggz + zv
    s16 = aggp + pv[:, 0:16]
    cnt = s16[:, KS]
    cv = pv[:, KS + 1]
    pos = jnp.sum(aggz * qv, axis=1) * INV_T / cnt
    lane = lax.broadcasted_iota(jnp.int32, (N, 16), 1)
    kf = jnp.where(lane <= 7, lane + 1, 1).astype(jnp.float32)
    sgn = jnp.where(lane % 2 == 0, 1.0, -1.0)
    coef = jnp.where(lane <= 7, sgn / kf, 0.0)
    linv = -jnp.log(cv)[:, None]
    rpow = jnp.exp(linv * kf)
    series = jnp.sum(s16 * coef * rpow, axis=1)
    neg = jnp.log(cv) + series / cnt
    loss = jnp.mean(-pos + LAMBDA_LOSS * neg)
    out_ref[...] = jnp.broadcast_to(loss, (8, 128))


def kernel(x, W0, b0, W1, b1, Wt1, bt1, gamma, beta, Wt2, bt2, Wp, bp,
           edge_index):
    src = edge_index[0]
    dst = edge_index[1]

    per_w = E // NW
    pad = jnp.full((NW, EPW - per_w), GARB, jnp.int32)
    srcp = jnp.concatenate([src.reshape(NW, per_w), pad], axis=1)
    srcp = srcp.reshape(NW, CH, 128)
    dstp = jnp.concatenate([dst.reshape(NW, per_w), pad], axis=1)
    dstp = dstp.reshape(NW, CH, 128)
    dst2 = jnp.where(src != dst, dst, GARB)
    dst2p = jnp.concatenate([dst2.reshape(NW, per_w), pad], axis=1)
    dst2p = dst2p.reshape(NW, CH, 128)

    degp = _degree_pass(srcp, dstp)
    degs = jnp.stack([degp[0, :, 0], degp[1, :, 0],
                      degp[0, :, 1], degp[1, :, 1]])

    y0 = _tc_call(_gcn_in_body,
                  jax.ShapeDtypeStruct((NPAD, D), jnp.float32),
                  x, W0, degs)
    qpad = _tc_call(_target_body,
                    jax.ShapeDtypeStruct((NPAD, D), jnp.float32),
                    x, Wt1, bt1, gamma, beta, Wt2, bt2, Wp, bp)

    agg1 = _rowagg_pass(y0, srcp, dstp)
    y1 = _tc_call(_gcn_mid_body,
                  jax.ShapeDtypeStruct((NPAD, D), jnp.float32),
                  agg1, degs, b0, W1)
    agg2 = _rowagg_pass(y1, srcp, dstp)
    zpad = _tc_call(_gcn_out_body,
                    jax.ShapeDtypeStruct((NPAD, D), jnp.float32),
                    agg2, degs, b1)

    ptab = _nxn_powers(zpad, qpad)
    aggz = _rowagg_pass(zpad, srcp, dst2p)
    ptab_b, _ = lax.optimization_barrier((ptab, aggz))
    aggp = _rowagg_pass(ptab_b, srcp, dst2p)

    loss = _tc_call(_loss_body,
                    jax.ShapeDtypeStruct((8, 128), jnp.float32),
                    aggz, aggp, ptab, zpad, qpad)
    return loss[0, 0]


# speedup vs baseline: 1.0948x; 1.0948x over previous
import functools

import jax
import jax.numpy as jnp
from jax import lax
from jax.experimental import pallas as pl
from jax.experimental.pallas import tpu as pltpu
from jax.experimental.pallas import tpu_sc as plsc

N = 10000
D = 128
TEMP = 0.5
INV_T = 1.0 / TEMP
LAM = 0.001
LAMBDA_LOSS = 1.0
E = 320000

NPAD = 10240
NOUT = 10240
GARB = N
KS = 8

NC = 2
NSUB = 16
NW = NC * NSUB
CH = 79
EPW = CH * 128
ROWS_PT = NOUT // NSUB

BR = 1024
BC = 1024

_sc_mesh = functools.partial(
    plsc.VectorSubcoreMesh, core_axis_name="c", subcore_axis_name="s")


def _degree_pass(srcp, dstp):
    @functools.partial(
        pl.kernel,
        out_type=jax.ShapeDtypeStruct((NC, NOUT, 128), jnp.float32),
        mesh=_sc_mesh(),
        scratch_types=[
            pltpu.VMEM((CH, 128), jnp.int32),
            pltpu.VMEM((CH, 128), jnp.int32),
            pltpu.VMEM((128, 128), jnp.float32),
            pltpu.VMEM((16, 128), jnp.float32),
            pltpu.VMEM_SHARED((NOUT, 128), jnp.float32),
            pltpu.SemaphoreType.DMA,
        ],
    )
    def deg(srcp_hbm, dstp_hbm, out_hbm, idxs_v, idxd_v, e_v, zb_v, acc_sh,
            sd):
        c = lax.axis_index("c")
        s = lax.axis_index("s")
        wid = c * NSUB + s
        pltpu.sync_copy(srcp_hbm.at[wid], idxs_v)
        pltpu.sync_copy(dstp_hbm.at[wid], idxd_v)
        lane = lax.broadcasted_iota(jnp.int32, (16,), 0)
        e0 = jnp.where(lane == 0, 1.0, 0.0).astype(jnp.float32)
        e1 = jnp.where(lane == 1, 1.0, 0.0).astype(jnp.float32)
        z16 = jnp.zeros((16,), jnp.float32)

        def fill0(i, carry):
            e_v[i, pl.ds(0, 16)] = e0
            for j in range(1, 8):
                e_v[i, pl.ds(j * 16, 16)] = z16
            return carry

        def fill1(i, carry):
            e_v[i, pl.ds(0, 16)] = e1
            return carry

        lax.fori_loop(0, 128, fill0, 0)
        for i in range(16):
            for j in range(8):
                zb_v[i, pl.ds(j * 16, 16)] = z16
        tb = s * ROWS_PT

        def zloop(r, carry):
            pltpu.sync_copy(zb_v, acc_sh.at[pl.ds(tb + r * 16, 16)])
            return carry

        lax.fori_loop(0, ROWS_PT // 16, zloop, 0)
        plsc.subcore_barrier()

        def mk_body(idx_v):
            def body(j, carry):
                pltpu.async_copy(e_v, acc_sh.at[idx_v.at[j]], sd, add=True)

                @pl.when(j >= 8)
                def _():
                    pltpu.make_async_copy(
                        e_v, acc_sh.at[idx_v.at[j - 8]], sd).wait()

                return carry
            return body

        def drain(idx_v):
            def body(j, carry):
                pltpu.make_async_copy(
                    e_v, acc_sh.at[idx_v.at[j]], sd).wait()
                return carry
            lax.fori_loop(CH - 8, CH, body, 0)

        lax.fori_loop(0, CH, mk_body(idxs_v), 0)
        drain(idxs_v)
        lax.fori_loop(0, 128, fill1, 0)
        lax.fori_loop(0, CH, mk_body(idxd_v), 0)
        drain(idxd_v)
        plsc.subcore_barrier()
        pltpu.sync_copy(acc_sh.at[pl.ds(tb, ROWS_PT)],
                        out_hbm.at[c, pl.ds(tb, ROWS_PT)])

    return deg(srcp, dstp)


NACC = 10112
RPT_A = NACC // NSUB

_AGG_SCRATCH = [
    pltpu.VMEM((1, 128), jnp.int32),
    pltpu.VMEM((1, 128), jnp.int32),
    pltpu.VMEM((1, 128), jnp.int32),
    pltpu.VMEM((1, 128), jnp.int32),
    pltpu.VMEM((1, 128), jnp.int32),
    pltpu.VMEM((1, 128), jnp.int32),
    pltpu.VMEM((128, 128), jnp.float32),
    pltpu.VMEM((128, 128), jnp.float32),
    pltpu.VMEM((128, 128), jnp.float32),
    pltpu.VMEM_SHARED((NACC, 128), jnp.float32),
] + [pltpu.SemaphoreType.DMA] * 12


def _agg_edges(table_hbm, srcp_hbm, dstp_hbm, wid, isl, dsl, rows, acc_sh,
               sg, sis, sdl, ss):
    pltpu.sync_copy(srcp_hbm.at[wid, pl.ds(0, 1)], isl[0])
    pltpu.sync_copy(srcp_hbm.at[wid, pl.ds(1, 1)], isl[1])
    pltpu.sync_copy(srcp_hbm.at[wid, pl.ds(2, 1)], isl[2])
    pltpu.sync_copy(dstp_hbm.at[wid, pl.ds(0, 1)], dsl[0])
    pltpu.sync_copy(dstp_hbm.at[wid, pl.ds(1, 1)], dsl[1])
    pltpu.async_copy(table_hbm.at[isl[0].at[0]], rows[0], sg[0])
    pltpu.async_copy(table_hbm.at[isl[1].at[0]], rows[1], sg[1])

    def third(j, a, b, c):
        del b
        pltpu.make_async_copy(table_hbm.at[isl[a].at[0]], rows[a],
                              sg[a]).wait()

        @pl.when(j + 3 < CH)
        def _():
            pltpu.async_copy(srcp_hbm.at[wid, pl.ds(j + 3, 1)], isl[a],
                             sis[a])

        @pl.when(j >= 2)
        def _():
            pltpu.make_async_copy(dstp_hbm.at[wid, pl.ds(j, 1)], dsl[a],
                                  sdl[a]).wait()

        pltpu.async_copy(rows[a], acc_sh.at[dsl[a].at[0]], ss[a], add=True)

        @pl.when(j >= 1)
        def _():
            pltpu.make_async_copy(rows[c], acc_sh.at[dsl[c].at[0]],
                                  ss[c]).wait()

        @pl.when(j + 2 < CH)
        def _():
            pltpu.async_copy(dstp_hbm.at[wid, pl.ds(j + 2, 1)], dsl[c],
                             sdl[c])

        @pl.when(jnp.logical_and(j >= 1, j + 2 < CH))
        def _():
            pltpu.make_async_copy(srcp_hbm.at[wid, pl.ds(j + 2, 1)], isl[c],
                                  sis[c]).wait()

        @pl.when(j + 2 < CH)
        def _():
            pltpu.async_copy(table_hbm.at[isl[c].at[0]], rows[c], sg[c])

    def body(j, carry):
        m = j % 3

        @pl.when(m == 0)
        def _():
            third(j, 0, 1, 2)

        @pl.when(m == 1)
        def _():
            third(j, 1, 2, 0)

        @pl.when(m == 2)
        def _():
            third(j, 2, 0, 1)

        return carry

    lax.fori_loop(0, CH, body, 0)
    last = (CH - 1) % 3
    pltpu.make_async_copy(rows[last], acc_sh.at[dsl[last].at[0]],
                          ss[last]).wait()


def _zero_acc(rows0_v, acc_sh, tb):
    z16 = jnp.zeros((16,), jnp.float32)

    def zfill(i, carry):
        for j in range(8):
            rows0_v[i, pl.ds(j * 16, 16)] = z16
        return carry

    lax.fori_loop(0, 128, zfill, 0)
    for i in range(4):
        pltpu.sync_copy(rows0_v, acc_sh.at[pl.ds(tb + i * 128, 128)])
    pltpu.sync_copy(rows0_v.at[pl.ds(0, 120)],
                    acc_sh.at[pl.ds(tb + 512, 120)])


def _rowagg_pass(table, srcp, dstp):
    @functools.partial(
        pl.kernel,
        out_type=jax.ShapeDtypeStruct((NC, NACC, 128), jnp.float32),
        mesh=_sc_mesh(),
        scratch_types=list(_AGG_SCRATCH),
    )
    def agg(table_hbm, srcp_hbm, dstp_hbm, out_hbm, i0, i1, i2, d0, d1, d2,
            r0, r1, r2, acc_sh, sg0, sg1, sg2, si0, si1, si2, sd0, sd1, sd2,
            ss0, ss1, ss2):
        c = lax.axis_index("c")
        s = lax.axis_index("s")
        wid = c * NSUB + s
        tb = s * RPT_A
        _zero_acc(r0, acc_sh, tb)
        plsc.subcore_barrier()
        _agg_edges(table_hbm, srcp_hbm, dstp_hbm, wid, (i0, i1, i2),
                   (d0, d1, d2), (r0, r1, r2), acc_sh, (sg0, sg1, sg2),
                   (si0, si1, si2), (sd0, sd1, sd2), (ss0, ss1, ss2))
        plsc.subcore_barrier()
        pltpu.sync_copy(acc_sh.at[pl.ds(tb, RPT_A)],
                        out_hbm.at[c, pl.ds(tb, RPT_A)])

    return agg(table, srcp, dstp)


def _tc_call(body, out_shape, *args):
    return pl.pallas_call(body, out_shape=out_shape)(*args)


def _gcn_in_body(x_ref, w_ref, degs_ref, out_ref):
    dego = degs_ref[0] + degs_ref[1]
    ns = lax.rsqrt(jnp.maximum(dego, 1.0))[:N]
    y = jnp.dot(x_ref[...], w_ref[...], preferred_element_type=jnp.float32)
    out_ref[0:N, :] = y * ns[:, None]
    out_ref[N:NPAD, :] = jnp.zeros((NPAD - N, D), jnp.float32)


def _target_body(x_ref, wt1_ref, bt1_ref, g_ref, be_ref, wt2_ref, bt2_ref,
                 wp_ref, bp_ref, out_ref):
    xv = x_ref[...]
    t = jnp.dot(xv, wt1_ref[...], preferred_element_type=jnp.float32)
    t = t + bt1_ref[...][None, :]
    mu = jnp.mean(t, axis=0)
    var = jnp.mean(t * t, axis=0) - mu * mu
    t = (t - mu[None, :]) * lax.rsqrt(var + 1e-5)[None, :]
    t = t * g_ref[...][None, :] + be_ref[...][None, :]
    t = jnp.maximum(t, 0.0)
    trans = jnp.dot(t, wt2_ref[...], preferred_element_type=jnp.float32)
    trans = trans + bt2_ref[...][None, :]
    p = jnp.dot(trans, wp_ref[...], preferred_element_type=jnp.float32)
    p = p + bp_ref[...][None, :]
    nrm = jnp.sqrt(jnp.sum(p * p, axis=1, keepdims=True))
    q = p / jnp.maximum(nrm, 1e-12)
    out_ref[0:N, :] = q
    out_ref[N:NPAD, :] = jnp.zeros((NPAD - N, D), jnp.float32)


def _gcn_mid_body(agg_ref, degs_ref, b0_ref, w1_ref, out_ref):
    dego = degs_ref[0] + degs_ref[1]
    degi = degs_ref[2] + degs_ref[3]
    ns = lax.rsqrt(jnp.maximum(dego, 1.0))[:N]
    nd = lax.rsqrt(jnp.maximum(degi, 1.0))[:N]
    a = (agg_ref[0] + agg_ref[1])[:N]
    h1 = jnp.maximum(a * nd[:, None] + b0_ref[...][None, :], 0.0)
    y = jnp.dot(h1, w1_ref[...], preferred_element_type=jnp.float32)
    out_ref[0:N, :] = y * ns[:, None]
    out_ref[N:NPAD, :] = jnp.zeros((NPAD - N, D), jnp.float32)


def _gcn_out_body(agg_ref, degs_ref, b1_ref, out_ref):
    degi = degs_ref[2] + degs_ref[3]
    nd = lax.rsqrt(jnp.maximum(degi, 1.0))[:N]
    a = (agg_ref[0] + agg_ref[1])[:N]
    h2 = a * nd[:, None] + b1_ref[...][None, :]
    nrm = jnp.sqrt(jnp.sum(h2 * h2, axis=1, keepdims=True))
    z = h2 / jnp.maximum(nrm, 1e-12)
    out_ref[0:N, :] = z
    out_ref[N:NPAD, :] = jnp.zeros((NPAD - N, D), jnp.float32)


def _nxn_body(zfull_ref, qfull_ref, zb_ref, v_ref):
    zb = zb_ref[...]

    def step(j, carry):
        a1, a2 = carry
        zc = zfull_ref[pl.ds(j * BC, BC), :]
        qc = qfull_ref[pl.ds(j * BC, BC), :]

        s1 = lax.dot_general(zb, zc, (((1,), (1,)), ((), ())),
                             preferred_element_type=jnp.float32)
        s2 = lax.dot_general(zb, qc, (((1,), (1,)), ((), ())),
                             preferred_element_type=jnp.float32)
        a1 = a1 + jnp.sum(jnp.exp(s1 * INV_T), axis=1)
        a2 = a2 + jnp.sum(jnp.exp(s2 * INV_T), axis=1)
        return a1, a2

    a1, a2 = lax.fori_loop(0, NPAD // BC, step,
                           (jnp.zeros((BR,), jnp.float32),
                            jnp.zeros((BR,), jnp.float32)))
    pad = float(NPAD - N)
    cc = a1 - pad
    aa = LAM * (a2 - pad)
    lane = lax.broadcasted_iota(jnp.int32, (BR, D), 1)
    kf = jnp.where(lane <= 7, lane + 1, 0).astype(jnp.float32)
    la = jnp.log(aa)[:, None]
    pw = jnp.exp(la * kf)
    val = jnp.where(
        lane <= 8, jnp.where(lane == 8, 1.0, pw),
        jnp.where(lane == 9, cc[:, None], 0.0))
    grow = lax.broadcasted_iota(jnp.int32, (BR, D), 0) + pl.program_id(0) * BR
    v_ref[...] = jnp.where(grow < N, val, 0.0)


def _nxn_powers(zpad, qpad):
    z16 = zpad.astype(jnp.bfloat16)
    q16 = qpad.astype(jnp.bfloat16)
    return pl.pallas_call(
        _nxn_body,
        grid=(NPAD // BR,),
        in_specs=[
            pl.BlockSpec((NPAD, D), lambda i: (0, 0)),
            pl.BlockSpec((NPAD, D), lambda i: (0, 0)),
            pl.BlockSpec((BR, D), lambda i: (i, 0)),
        ],
        out_specs=pl.BlockSpec((BR, D), lambda i: (i, 0)),
        out_shape=jax.ShapeDtypeStruct((NPAD, D), jnp.float32),
    )(z16, q16, z16)


def _loss_body(aggz_ref, aggp_ref, p_ref, z_ref, q_ref, out_ref):
    aggz = (aggz_ref[0] + aggz_ref[1])[:N]
    aggp = (aggp_ref[0] + aggp_ref[1])[:N, 0:16]
    pv = p_ref[...][:N]
    zv = z_ref[...][:N]
    qv = q_ref[...][:N]
    aggz = aggz + zv
    s16 = aggp + pv[:, 0:16]
    cnt = s16[:, KS]
    cv = pv[:, KS + 1]
    pos = jnp.sum(aggz * qv, axis=1) * INV_T / cnt
    lane = lax.broadcasted_iota(jnp.int32, (N, 16), 1)
    kf = jnp.where(lane <= 7, lane + 1, 1).astype(jnp.float32)
    sgn = jnp.where(lane % 2 == 0, 1.0, -1.0)
    coef = jnp.where(lane <= 7, sgn / kf, 0.0)
    linv = -jnp.log(cv)[:, None]
    rpow = jnp.exp(linv * kf)
    series = jnp.sum(s16 * coef * rpow, axis=1)
    neg = jnp.log(cv) + series / cnt
    loss = jnp.mean(-pos + LAMBDA_LOSS * neg)
    out_ref[...] = jnp.broadcast_to(loss, (8, 128))


def kernel(x, W0, b0, W1, b1, Wt1, bt1, gamma, beta, Wt2, bt2, Wp, bp,
           edge_index):
    src = edge_index[0]
    dst = edge_index[1]

    per_w = E // NW
    padg = jnp.full((NW, EPW - per_w), GARB, jnp.int32)
    pad0 = jnp.zeros((NW, EPW - per_w), jnp.int32)
    srcp = jnp.concatenate([src.reshape(NW, per_w), padg], axis=1)
    srcp = srcp.reshape(NW, CH, 128)
    dstpg = jnp.concatenate([dst.reshape(NW, per_w), padg], axis=1)
    dstpg = dstpg.reshape(NW, CH, 128)
    dstp = jnp.concatenate([dst.reshape(NW, per_w), pad0], axis=1)
    dstp = dstp.reshape(NW, CH, 128)
    mask = src != dst
    src3 = jnp.where(mask, src, GARB)
    src3p = jnp.concatenate([src3.reshape(NW, per_w), padg], axis=1)
    src3p = src3p.reshape(NW, CH, 128)
    dst3 = jnp.where(mask, dst, 0)
    dst3p = jnp.concatenate([dst3.reshape(NW, per_w), pad0], axis=1)
    dst3p = dst3p.reshape(NW, CH, 128)

    degp = _degree_pass(srcp, dstpg)
    degs = jnp.stack([degp[0, :, 0], degp[1, :, 0],
                      degp[0, :, 1], degp[1, :, 1]])

    y0 = _tc_call(_gcn_in_body,
                  jax.ShapeDtypeStruct((NPAD, D), jnp.float32),
                  x, W0, degs)
    qpad = _tc_call(_target_body,
                    jax.ShapeDtypeStruct((NPAD, D), jnp.float32),
                    x, Wt1, bt1, gamma, beta, Wt2, bt2, Wp, bp)

    agg1 = _rowagg_pass(y0, srcp, dstp)
    y1 = _tc_call(_gcn_mid_body,
                  jax.ShapeDtypeStruct((NPAD, D), jnp.float32),
                  agg1, degs, b0, W1)
    agg2 = _rowagg_pass(y1, srcp, dstp)
    zpad = _tc_call(_gcn_out_body,
                    jax.ShapeDtypeStruct((NPAD, D), jnp.float32),
                    agg2, degs, b1)

    ptab = _nxn_powers(zpad, qpad)
    aggz = _rowagg_pass(zpad, src3p, dst3p)
    ptab_b, _ = lax.optimization_barrier((ptab, aggz))
    aggp = _rowagg_pass(ptab_b, src3p, dst3p)

    loss = _tc_call(_loss_body,
                    jax.ShapeDtypeStruct((8, 128), jnp.float32),
                    aggz, aggp, ptab, zpad, qpad)
    return loss[0, 0]
